# baseline (speedup 1.0000x reference)
import dataclasses
import functools

import jax
import jax.numpy as jnp
from jax import lax
from jax.experimental import pallas as pl
from jax.experimental.pallas import tpu as pltpu
from jax.experimental.pallas import tpu_sc as plsc

B = 16384
D = 64
K = 20
NC = 2
NS = 16
NW = NC * NS
BPW = B // NW
W = 64
NCHUNK = BPW // W
NPAIR = W * (K + 1)
GATHER_ROWS = 128
NEG_GATHERS = W * K // GATHER_ROWS


def _logsig(x):
    m = jnp.minimum(x, 0.0)
    u = jnp.exp(-jnp.abs(x))
    t = u / (2.0 + u)
    t2 = t * t
    ln1p = 2.0 * t * (1.0 + t2 * (1.0 / 3.0 + t2 * (0.2 + t2 * (1.0 / 7.0 + t2 * (1.0 / 9.0)))))
    return m - ln1p


def _sc_partials(center2d, pos2d, neg2d, center_w, context_w):
    mesh = plsc.VectorSubcoreMesh(core_axis_name="c", subcore_axis_name="s")
    cp = pltpu.CompilerParams()
    fields = pltpu.CompilerParams.__dataclass_fields__
    if "needs_layout_passes" in fields:
        cp = dataclasses.replace(cp, needs_layout_passes=False)
    if "use_tc_tiling_on_sc" in fields:
        cp = dataclasses.replace(cp, use_tc_tiling_on_sc=False)

    @functools.partial(
        pl.kernel,
        out_type=jax.ShapeDtypeStruct((NW, 16), jnp.float32),
        mesh=mesh,
        compiler_params=cp,
        scratch_types=[
            pltpu.VMEM((NCHUNK, W), jnp.int32),
            pltpu.VMEM((NCHUNK, W), jnp.int32),
            pltpu.VMEM((NCHUNK * NEG_GATHERS, GATHER_ROWS), jnp.int32),
            pltpu.VMEM((W, D), jnp.float32),
            pltpu.VMEM((W, D), jnp.float32),
            pltpu.VMEM((W * K, D), jnp.float32),
            pltpu.VMEM((NPAIR,), jnp.float32),
            pltpu.VMEM((16,), jnp.float32),
            pltpu.SemaphoreType.DMA,
        ],
    )
    def body(center_hbm, pos_hbm, neg_hbm, cw_hbm, ctw_hbm, out_hbm,
             cidx, pidx, nidx, crows, prows, nrows, scores, accv, sem):
        wid = lax.axis_index("s") * NC + lax.axis_index("c")

        pltpu.sync_copy(center_hbm.at[pl.ds(wid * NCHUNK, NCHUNK)], cidx)
        pltpu.sync_copy(pos_hbm.at[pl.ds(wid * NCHUNK, NCHUNK)], pidx)
        nrow0 = wid * (NCHUNK * NEG_GATHERS)
        pltpu.sync_copy(neg_hbm.at[pl.ds(nrow0, NCHUNK * NEG_GATHERS)], nidx)

        lane0 = lax.iota(jnp.int32, 16) == 0

        def write_score(i, val):
            plsc.store_scatter(scores, [jnp.full((16,), i, jnp.int32)],
                               jnp.full((16,), val, jnp.float32), mask=lane0)

        loss = jnp.zeros((16,), jnp.float32)
        for j in range(NCHUNK):
            copies = [
                pltpu.async_copy(cw_hbm.at[cidx.at[j]], crows, sem),
                pltpu.async_copy(ctw_hbm.at[pidx.at[j]], prows, sem),
            ]
            for t in range(NEG_GATHERS):
                copies.append(pltpu.async_copy(
                    ctw_hbm.at[nidx.at[j * NEG_GATHERS + t]],
                    nrows.at[pl.ds(t * GATHER_ROWS, GATHER_ROWS)], sem))
            for cp in copies:
                cp.wait()

            def pair_dots(b, carry):
                c0 = crows[b, 0:16]
                c1 = crows[b, 16:32]
                c2 = crows[b, 32:48]
                c3 = crows[b, 48:64]
                acc = (c0 * prows[b, 0:16] + c1 * prows[b, 16:32]
                       + c2 * prows[b, 32:48] + c3 * prows[b, 48:64])
                write_score(b * (K + 1), jnp.sum(acc))
                for k in range(K):
                    r = b * K + k
                    acc = (c0 * nrows[r, 0:16] + c1 * nrows[r, 16:32]
                           + c2 * nrows[r, 32:48] + c3 * nrows[r, 48:64])
                    write_score(b * (K + 1) + 1 + k, -jnp.sum(acc))
                return carry

            lax.fori_loop(0, W, pair_dots, 0)

            def logsig_acc(v, lacc):
                return lacc + _logsig(scores[pl.ds(v * 16, 16)])

            loss = lax.fori_loop(0, NPAIR // 16, logsig_acc, loss)

        accv[...] = loss
        pltpu.sync_copy(accv, out_hbm.at[wid])

    return body(center2d, pos2d, neg2d, center_w, context_w)


def _tc_finish(partials):
    def body(x_ref, o_ref):
        o_ref[0, 0] = -jnp.sum(x_ref[...]) / jnp.float32(B)

    return pl.pallas_call(
        body,
        out_shape=jax.ShapeDtypeStruct((1, 1), jnp.float32),
        out_specs=pl.BlockSpec(memory_space=pltpu.SMEM),
    )(partials)


def kernel(center, pos_ctx, neg_ctx, center_w, context_w):
    center2d = center.astype(jnp.int32).reshape(NW * NCHUNK, W)
    pos2d = pos_ctx.astype(jnp.int32).reshape(NW * NCHUNK, W)
    neg2d = neg_ctx.astype(jnp.int32).reshape(NW * NCHUNK * NEG_GATHERS, GATHER_ROWS)
    partials = _sc_partials(center2d, pos2d, neg2d, center_w, context_w)
    return _tc_finish(partials)[0, 0]
